# baseline (speedup 1.0000x reference)
import functools

import jax
import jax.numpy as jnp
from jax import lax
from jax.experimental import pallas as pl
from jax.experimental.pallas import tpu as pltpu
from jax.experimental.pallas import tpu_sc as plsc

VOCAB = 1000000
D = 64
DP = 128
B = 4096 * 200
NC, NS = 2, 16
NW = NC * NS
B_PER_W = B // NW
CH = 128
NBUF = 4
N_CH = B_PER_W // CH
NG = N_CH // NBUF

_mesh = plsc.VectorSubcoreMesh(
    core_axis_name="c", subcore_axis_name="s", num_cores=NC, num_subcores=NS
)


@functools.partial(
    pl.kernel,
    out_type=jax.ShapeDtypeStruct((B, D), jnp.float32),
    mesh=_mesh,
    compiler_params=pltpu.CompilerParams(use_tc_tiling_on_sc=False),
    scratch_types=[
        pltpu.VMEM((B_PER_W,), jnp.int32),
        pltpu.VMEM((NBUF, CH, DP), jnp.float32),
    ]
    + [pltpu.SemaphoreType.DMA] * NBUF
    + [pltpu.SemaphoreType.DMA] * NBUF,
)
def _embed_sc(x_hbm, table_hbm, out_hbm, idx_v, rows_v, *sems):
    gsems = sems[:NBUF]
    osems = sems[NBUF:]
    wid = lax.axis_index("s") * NC + lax.axis_index("c")
    base = pl.multiple_of(wid * B_PER_W, B_PER_W)

    pltpu.sync_copy(x_hbm.at[pl.ds(base, B_PER_W)], idx_v)

    def group(g, _):
        for b in range(NBUF):
            c = g * NBUF + b
            off = pl.multiple_of(base + c * CH, CH)

            @pl.when(g > 0)
            def _drain():
                pltpu.make_async_copy(
                    rows_v.at[b, :, pl.ds(0, D)], out_hbm.at[pl.ds(off, CH)], osems[b]
                ).wait()

            pltpu.make_async_copy(
                table_hbm.at[idx_v.at[pl.ds(c * CH, CH)]],
                rows_v.at[b],
                gsems[b],
            ).start()

        for b in range(NBUF):
            c = g * NBUF + b
            off = pl.multiple_of(base + c * CH, CH)
            pltpu.make_async_copy(
                table_hbm.at[idx_v.at[pl.ds(c * CH, CH)]],
                rows_v.at[b],
                gsems[b],
            ).wait()
            pltpu.make_async_copy(
                rows_v.at[b, :, pl.ds(0, D)], out_hbm.at[pl.ds(off, CH)], osems[b]
            ).start()
        return _

    lax.fori_loop(0, NG, group, 0)

    for b in range(NBUF):
        c = (NG - 1) * NBUF + b
        off = pl.multiple_of(base + c * CH, CH)
        pltpu.make_async_copy(
            rows_v.at[b, :, pl.ds(0, D)], out_hbm.at[pl.ds(off, CH)], osems[b]
        ).wait()


@jax.jit
def kernel(x, table):
    xt = jnp.swapaxes(x, 0, 1).reshape(-1)
    tp = jnp.pad(table, ((0, 0), (0, DP - D)))
    out = _embed_sc(xt, tp)
    return jnp.swapaxes(out.reshape(x.shape[1], x.shape[0], D), 0, 1)


# speedup vs baseline: 1.0763x; 1.0763x over previous
import functools

import jax
import jax.numpy as jnp
from jax import lax
from jax.experimental import pallas as pl
from jax.experimental.pallas import tpu as pltpu
from jax.experimental.pallas import tpu_sc as plsc

VOCAB = 1000000
D = 64
HIST = 200
BATCH = 4096
B = BATCH * HIST
NC, NS = 2, 16
NW = NC * NS
B_PER_W = B // NW
CH = 128
NBUF = 4
N_CH = B_PER_W // CH
NG = N_CH // NBUF

_mesh = plsc.VectorSubcoreMesh(
    core_axis_name="c", subcore_axis_name="s", num_cores=NC, num_subcores=NS
)


@functools.partial(
    pl.kernel,
    out_type=jax.ShapeDtypeStruct((HIST, BATCH, D), jnp.float32),
    mesh=_mesh,
    compiler_params=pltpu.CompilerParams(use_tc_tiling_on_sc=False),
    scratch_types=[
        pltpu.VMEM((B_PER_W,), jnp.int32),
        pltpu.VMEM((NBUF, CH, D), jnp.float32),
    ]
    + [pltpu.SemaphoreType.DMA] * NBUF
    + [pltpu.SemaphoreType.DMA] * NBUF,
)
def _embed_sc(x_hbm, table_hbm, out_hbm, idx_v, rows_v, *sems):
    gsems = sems[:NBUF]
    osems = sems[NBUF:]
    wid = lax.axis_index("s") * NC + lax.axis_index("c")
    base = pl.multiple_of(wid * B_PER_W, B_PER_W)

    pltpu.sync_copy(x_hbm.at[pl.ds(base, B_PER_W)], idx_v)

    def out_dst(c):
        off = base + c * CH
        t = off // BATCH
        b0 = pl.multiple_of(off - t * BATCH, CH)
        return out_hbm.at[t, pl.ds(b0, CH)]

    def group(g, _):
        for bf in range(NBUF):
            c = g * NBUF + bf

            @pl.when(g > 0)
            def _drain():
                pltpu.make_async_copy(
                    rows_v.at[bf], out_dst(c - NBUF), osems[bf]
                ).wait()

            pltpu.make_async_copy(
                table_hbm.at[idx_v.at[pl.ds(c * CH, CH)]],
                rows_v.at[bf],
                gsems[bf],
            ).start()

        for bf in range(NBUF):
            c = g * NBUF + bf
            pltpu.make_async_copy(
                table_hbm.at[idx_v.at[pl.ds(c * CH, CH)]],
                rows_v.at[bf],
                gsems[bf],
            ).wait()
            pltpu.make_async_copy(
                rows_v.at[bf], out_dst(c), osems[bf]
            ).start()
        return _

    lax.fori_loop(0, NG, group, 0)

    for bf in range(NBUF):
        c = (NG - 1) * NBUF + bf
        pltpu.make_async_copy(rows_v.at[bf], out_dst(c), osems[bf]).wait()


@jax.jit
def kernel(x, table):
    xt = jnp.swapaxes(x, 0, 1).reshape(-1)
    out = _embed_sc(xt, table)
    return jnp.swapaxes(out, 0, 1)


# speedup vs baseline: 1.0793x; 1.0028x over previous
import functools

import jax
import jax.numpy as jnp
from jax import lax
from jax.experimental import pallas as pl
from jax.experimental.pallas import tpu as pltpu
from jax.experimental.pallas import tpu_sc as plsc

VOCAB = 1000000
D = 64
HIST = 200
BATCH = 4096
B = BATCH * HIST
NC, NS = 2, 16
NW = NC * NS
B_PER_W = B // NW
CH = 128
NBUF = 8
N_CH = B_PER_W // CH
NG = N_CH // NBUF

_mesh = plsc.VectorSubcoreMesh(
    core_axis_name="c", subcore_axis_name="s", num_cores=NC, num_subcores=NS
)


@functools.partial(
    pl.kernel,
    out_type=jax.ShapeDtypeStruct((HIST, BATCH, D), jnp.float32),
    mesh=_mesh,
    compiler_params=pltpu.CompilerParams(use_tc_tiling_on_sc=False),
    scratch_types=[
        pltpu.VMEM((B_PER_W,), jnp.int32),
        pltpu.VMEM((NBUF, CH, D), jnp.float32),
    ]
    + [pltpu.SemaphoreType.DMA] * NBUF
    + [pltpu.SemaphoreType.DMA] * NBUF,
)
def _embed_sc(x_hbm, table_hbm, out_hbm, idx_v, rows_v, *sems):
    gsems = sems[:NBUF]
    osems = sems[NBUF:]
    wid = lax.axis_index("s") * NC + lax.axis_index("c")
    base = pl.multiple_of(wid * B_PER_W, B_PER_W)

    pltpu.sync_copy(x_hbm.at[pl.ds(base, B_PER_W)], idx_v)

    def out_dst(c):
        off = base + c * CH
        t = off // BATCH
        b0 = pl.multiple_of(off - t * BATCH, CH)
        return out_hbm.at[t, pl.ds(b0, CH)]

    def group(g, _):
        for bf in range(NBUF):
            c = g * NBUF + bf

            @pl.when(g > 0)
            def _drain():
                pltpu.make_async_copy(
                    rows_v.at[bf], out_dst(c - NBUF), osems[bf]
                ).wait()

            pltpu.make_async_copy(
                table_hbm.at[idx_v.at[pl.ds(c * CH, CH)]],
                rows_v.at[bf],
                gsems[bf],
            ).start()

        for bf in range(NBUF):
            c = g * NBUF + bf
            pltpu.make_async_copy(
                table_hbm.at[idx_v.at[pl.ds(c * CH, CH)]],
                rows_v.at[bf],
                gsems[bf],
            ).wait()
            pltpu.make_async_copy(
                rows_v.at[bf], out_dst(c), osems[bf]
            ).start()
        return _

    lax.fori_loop(0, NG, group, 0)

    for bf in range(NBUF):
        c = (NG - 1) * NBUF + bf
        pltpu.make_async_copy(rows_v.at[bf], out_dst(c), osems[bf]).wait()


@jax.jit
def kernel(x, table):
    xt = jnp.swapaxes(x, 0, 1).reshape(-1)
    out = _embed_sc(xt, table)
    return jnp.swapaxes(out, 0, 1)
